# baseline (speedup 1.0000x reference)
import functools

import jax
import jax.numpy as jnp
from jax import lax
from jax.experimental import pallas as pl
from jax.experimental.pallas import tpu as pltpu
from jax.experimental.pallas import tpu_sc as plsc

BATCH = 16384
DIM = 64
LANES = 16
NUM_CORES = 2
NUM_SUBCORES = 16
NUM_WORKERS = NUM_CORES * NUM_SUBCORES
SW = 1024
N_GROUPS = BATCH // LANES
UEMB_ROWS = BATCH + LANES
DUMMY_PV = 63 << 24

NU = 100000
NI = 1000000
NS_U = (NU + SW - 1) // SW
NS_I = (NI + SW - 1) // SW
KMAX_U = (NS_U + NUM_SUBCORES - 1) // NUM_SUBCORES
KMAX_I = (NS_I + NUM_WORKERS - 1) // NUM_WORKERS


def _body(uids_hbm, iids_hbm, ut_hbm, it_hbm, out2_hbm, uemb_hbm, part_hbm,
          ids, hits1, strip, ustage, out_v, row_buf, acc_v, sem):
    c = lax.axis_index("c")
    s = lax.axis_index("s")
    w = s * NUM_CORES + c
    lane = lax.iota(jnp.int32, LANES)
    uemb_base = c * UEMB_ROWS

    def scan_hits(owner, stride, ls_shift):
        def g_body(g, cnt):
            idg = ids[pl.ds(g * LANES, LANES)]
            sid = jnp.right_shift(idg, 10)
            mask = jnp.bitwise_and(sid, stride - 1) == owner
            pv = (jnp.left_shift(jnp.right_shift(sid, ls_shift), 24)
                  | jnp.left_shift(jnp.bitwise_and(idg, SW - 1), 14)
                  | (g * LANES + lane))
            plsc.store_compressed(hits1.at[pl.ds(cnt, LANES)], pv, mask=mask)
            return cnt + plsc.all_reduce_population_count(mask)[0]

        cnt = lax.fori_loop(0, N_GROUPS, g_body, 0)
        hits1[pl.ds(cnt, LANES)] = jnp.full((LANES,), DUMMY_PV, jnp.int32)
        return jnp.right_shift(cnt + LANES - 1, 4)

    def run_phase(tab_hbm, rows, n_strips, owner, stride, ls_shift,
                  kmax, phase):
        n1g = scan_hits(owner, stride, ls_shift)

        def strip_body(k, _):
            t = owner + stride * k

            @pl.when(t < n_strips)
            def _():
                base = pl.multiple_of(jnp.minimum(t * SW, rows - SW), 8)
                delta = t * SW - base
                pltpu.sync_copy(tab_hbm.at[:, pl.ds(base, SW)], strip)

                def g2_body(g2, _2):
                    pv = hits1[pl.ds(g2 * LANES, LANES)]
                    mask = jnp.right_shift(pv, 24) == k
                    pop = plsc.all_reduce_population_count(mask)[0]

                    @pl.when(pop > 0)
                    def _3():
                        offv = jnp.where(
                            mask,
                            jnp.bitwise_and(jnp.right_shift(pv, 14), SW - 1)
                            + delta, 0)
                        posv = jnp.bitwise_and(pv, BATCH - 1)
                        pos_flat = uemb_base + jnp.where(
                            mask, posv, BATCH + lane)
                        if phase == 1:
                            for d in range(DIM):
                                dv = jnp.full((LANES,), d, jnp.int32)
                                vals = plsc.load_gather(strip, [dv, offv])
                                plsc.store_scatter(ustage, [lane, dv], vals)
                            pltpu.async_copy(
                                ustage, uemb_hbm.at[pos_flat], sem).wait()
                        else:
                            pltpu.async_copy(
                                uemb_hbm.at[pos_flat], ustage, sem).wait()
                            acc = jnp.zeros((LANES,), jnp.float32)
                            for d in range(DIM):
                                dv = jnp.full((LANES,), d, jnp.int32)
                                u = plsc.load_gather(ustage, [lane, dv])
                                v = plsc.load_gather(strip, [dv, offv])
                                acc = acc + u * v
                            plsc.store_scatter(out_v, [posv], acc, mask=mask)
                    return 0

                lax.fori_loop(0, n1g, g2_body, 0)
            return 0

        lax.fori_loop(0, kmax, strip_body, 0)

    pltpu.sync_copy(uids_hbm, ids)
    run_phase(ut_hbm, NU, NS_U, s, NUM_SUBCORES, 4, KMAX_U, 1)
    plsc.subcore_barrier()

    def zero_body(g, _):
        out_v[pl.ds(g * LANES, LANES)] = jnp.zeros((LANES,), jnp.float32)
        return 0
    lax.fori_loop(0, N_GROUPS, zero_body, 0)
    pltpu.sync_copy(iids_hbm, ids)
    run_phase(it_hbm, NI, NS_I, w, NUM_WORKERS, 5, KMAX_I, 2)

    pltpu.sync_copy(out_v, part_hbm.at[c, s])
    plsc.subcore_barrier()
    col0 = pl.multiple_of(s * SW, 8)

    def zacc_body(g, _):
        acc_v[pl.ds(g * LANES, LANES)] = jnp.zeros((LANES,), jnp.float32)
        return 0
    lax.fori_loop(0, SW // LANES, zacc_body, 0)
    for r in range(NUM_SUBCORES):
        pltpu.sync_copy(part_hbm.at[c, r, pl.ds(col0, SW)], row_buf)

        def add_body(g, _):
            sl = pl.ds(g * LANES, LANES)
            acc_v[sl] = acc_v[sl] + row_buf[sl]
            return 0
        lax.fori_loop(0, SW // LANES, add_body, 0)
    pltpu.sync_copy(acc_v, out2_hbm.at[c, pl.ds(col0, SW)])


def kernel(user_ids, item_ids, user_table, item_table):
    ut = user_table.T
    it = item_table.T
    uids = user_ids.astype(jnp.int32)
    iids = item_ids.astype(jnp.int32)

    mesh = plsc.VectorSubcoreMesh(
        core_axis_name="c", subcore_axis_name="s",
        num_cores=NUM_CORES, num_subcores=NUM_SUBCORES)

    run = pl.kernel(
        _body,
        out_type=[
            jax.ShapeDtypeStruct((NUM_CORES, BATCH), jnp.float32),
            jax.ShapeDtypeStruct((NUM_CORES * UEMB_ROWS, DIM), jnp.float32),
            jax.ShapeDtypeStruct((NUM_CORES, NUM_SUBCORES, BATCH),
                                 jnp.float32),
        ],
        mesh=mesh,
        scratch_types=[
            pltpu.VMEM((BATCH,), jnp.int32),
            pltpu.VMEM((BATCH + LANES,), jnp.int32),
            pltpu.VMEM((DIM, SW), jnp.float32),
            pltpu.VMEM((LANES, DIM), jnp.float32),
            pltpu.VMEM((BATCH,), jnp.float32),
            pltpu.VMEM((SW,), jnp.float32),
            pltpu.VMEM((SW,), jnp.float32),
            pltpu.SemaphoreType.DMA,
        ],
        compiler_params=pltpu.CompilerParams(
            needs_layout_passes=False, use_tc_tiling_on_sc=False),
    )
    out2, _, _ = run(uids, iids, ut, it)
    return out2[0] + out2[1]


# speedup vs baseline: 8.9290x; 8.9290x over previous
import functools

import jax
import jax.numpy as jnp
from jax import lax
from jax.experimental import pallas as pl
from jax.experimental.pallas import tpu as pltpu
from jax.experimental.pallas import tpu_sc as plsc

BATCH = 16384
DIM = 64
LANES = 16
NUM_CORES = 2
NUM_SUBCORES = 16
NUM_WORKERS = NUM_CORES * NUM_SUBCORES
B_PER_W = BATCH // NUM_WORKERS
IDX_W = 128
N_GATHER = B_PER_W // IDX_W
GROUPS = B_PER_W // LANES


def _body(uids_hbm, iids_hbm, user_hbm, item_hbm, out_hbm,
          idx_u, idx_i, rows_u, rows_i, out_v, sem):
    w = lax.axis_index("s") * NUM_CORES + lax.axis_index("c")
    base = w * B_PER_W

    pltpu.sync_copy(uids_hbm.at[pl.ds(w * N_GATHER, N_GATHER)], idx_u)
    pltpu.sync_copy(iids_hbm.at[pl.ds(w * N_GATHER, N_GATHER)], idx_i)

    copies = []
    for j in range(N_GATHER):
        copies.append(pltpu.async_copy(
            user_hbm.at[idx_u.at[j]], rows_u.at[pl.ds(j * IDX_W, IDX_W)], sem))
        copies.append(pltpu.async_copy(
            item_hbm.at[idx_i.at[j]], rows_i.at[pl.ds(j * IDX_W, IDX_W)], sem))
    for c in copies:
        c.wait()

    lane = lax.iota(jnp.int32, LANES)

    def group(g, _):
        rb = g * LANES
        row_idx = rb + lane
        acc = jnp.zeros((LANES,), jnp.float32)
        for d in range(DIM):
            col = jnp.full((LANES,), d, jnp.int32)
            u = plsc.load_gather(rows_u, [row_idx, col])
            v = plsc.load_gather(rows_i, [row_idx, col])
            acc = acc + u * v
        out_v[pl.ds(rb, LANES)] = acc
        return 0

    lax.fori_loop(0, GROUPS, group, 0)

    pltpu.sync_copy(out_v, out_hbm.at[pl.ds(base, B_PER_W)])


def kernel(user_ids, item_ids, user_table, item_table):
    uids = user_ids.astype(jnp.int32).reshape(NUM_WORKERS * N_GATHER, IDX_W)
    iids = item_ids.astype(jnp.int32).reshape(NUM_WORKERS * N_GATHER, IDX_W)

    mesh = plsc.VectorSubcoreMesh(
        core_axis_name="c", subcore_axis_name="s",
        num_cores=NUM_CORES, num_subcores=NUM_SUBCORES)

    run = pl.kernel(
        _body,
        out_type=jax.ShapeDtypeStruct((BATCH,), jnp.float32),
        mesh=mesh,
        scratch_types=[
            pltpu.VMEM((N_GATHER, IDX_W), jnp.int32),
            pltpu.VMEM((N_GATHER, IDX_W), jnp.int32),
            pltpu.VMEM((B_PER_W, DIM), jnp.float32),
            pltpu.VMEM((B_PER_W, DIM), jnp.float32),
            pltpu.VMEM((B_PER_W,), jnp.float32),
            pltpu.SemaphoreType.DMA,
        ],
        compiler_params=pltpu.CompilerParams(
            needs_layout_passes=False, use_tc_tiling_on_sc=False),
    )
    return run(uids, iids, user_table, item_table)
